# baseline (speedup 1.0000x reference)
import functools

import jax
import jax.numpy as jnp
from jax import lax
from jax.experimental import pallas as pl
from jax.experimental.pallas import tpu as pltpu
from jax.experimental.pallas import tpu_sc as plsc

EMBED = 32
POS_V = 64
BATCH = 4096
SEQ = 128

NC, NS = 2, 16
NW = NC * NS
SEQ_PER_W = BATCH // NW
CHUNK = 4
NCHUNK = SEQ_PER_W // CHUNK
NBUF = 4
NLANE = 16
SBLK = SEQ // NLANE


def _build_post(pos_stage, post_v):
    lanes = lax.iota(jnp.int32, NLANE)

    @plsc.parallel_loop(0, SBLK)
    def body(sb):
        kvec = jnp.minimum(lanes + sb * NLANE, POS_V - 1)
        for e in range(EMBED):
            evec = jnp.full((NLANE,), e, jnp.int32)
            post_v[e, pl.ds(sb * NLANE, NLANE)] = plsc.load_gather(
                pos_stage, [kvec, evec])


def _transpose_add(rows_v, trows_v, post_v):
    lanes = lax.iota(jnp.int32, NLANE)

    @plsc.parallel_loop(0, NLANE)
    def body(d):
        rot = jnp.bitwise_and(lanes + d, NLANE - 1)
        for h in range(EMBED // NLANE):
            evec = lanes + h * NLANE
            for sb in range(SBLK):
                svec = rot + sb * NLANE
                p = plsc.load_gather(post_v, [evec, svec])
                for j in range(CHUNK):
                    x = plsc.load_gather(rows_v.at[j], [svec, evec]) + p
                    plsc.store_scatter(trows_v.at[j], [evec, svec], x)


def _sc_body(tok_hbm, pos_hbm, patch_hbm, out_hbm, idx_v, pos_stage,
             rows_v, post_v, trows_v, gsems, ssems):
    wid = lax.axis_index("s") * NC + lax.axis_index("c")
    seq0 = wid * SEQ_PER_W

    pltpu.sync_copy(patch_hbm.at[pl.ds(seq0, SEQ_PER_W)], idx_v)
    pltpu.sync_copy(pos_hbm, pos_stage)
    _build_post(pos_stage, post_v)

    def issue_gathers(c, b):
        for j in range(CHUNK):
            pltpu.async_copy(
                tok_hbm.at[idx_v.at[c * CHUNK + j]],
                rows_v.at[b, j],
                gsems.at[b],
            )

    def wait_gathers(c, b):
        pltpu.make_async_copy(
            out_hbm.at[pl.ds(seq0, CHUNK)],
            rows_v.at[b],
            gsems.at[b],
        ).wait()

    issue_gathers(0, 0)
    issue_gathers(1, 1)

    def outer(c4, _):
        for u in range(NBUF):
            c = c4 * NBUF + u
            tb = u % 2

            @pl.when(c + 2 < NCHUNK)
            def _():
                issue_gathers(c + 2, (u + 2) % NBUF)

            wait_gathers(c, u)

            if u >= 2:
                _wait_store(out_hbm, trows_v, ssems, seq0, tb)
            else:
                @pl.when(c4 >= 1)
                def _():
                    _wait_store(out_hbm, trows_v, ssems, seq0, tb)

            _transpose_add(rows_v.at[u], trows_v.at[tb], post_v)
            pltpu.async_copy(
                trows_v.at[tb],
                out_hbm.at[pl.ds(seq0 + c * CHUNK, CHUNK)],
                ssems.at[tb],
            )
        return 0

    lax.fori_loop(0, NCHUNK // NBUF, outer, 0, unroll=False)
    for tb in range(2):
        _wait_store(out_hbm, trows_v, ssems, seq0, tb)


def _wait_store(out_hbm, trows_v, ssems, seq0, tb):
    pltpu.make_async_copy(
        trows_v.at[tb],
        out_hbm.at[pl.ds(seq0, CHUNK)],
        ssems.at[tb],
    ).wait()


@functools.partial(
    pl.kernel,
    out_type=jax.ShapeDtypeStruct((BATCH, EMBED, SEQ), jnp.float32),
    mesh=plsc.VectorSubcoreMesh(core_axis_name="c", subcore_axis_name="s"),
    scratch_types=[
        pltpu.VMEM((SEQ_PER_W, SEQ), jnp.int32),
        pltpu.VMEM((POS_V, EMBED), jnp.float32),
        pltpu.VMEM((NBUF, CHUNK, SEQ, EMBED), jnp.float32),
        pltpu.VMEM((EMBED, SEQ), jnp.float32),
        pltpu.VMEM((2, CHUNK, EMBED, SEQ), jnp.float32),
        pltpu.SemaphoreType.DMA((NBUF,)),
        pltpu.SemaphoreType.DMA((2,)),
    ],
    compiler_params=pltpu.CompilerParams(
        use_tc_tiling_on_sc=False, needs_layout_passes=False),
)
def _sc_embed(tok_hbm, pos_hbm, patch_hbm, out_hbm, idx_v, pos_stage,
              rows_v, post_v, trows_v, gsems, ssems):
    _sc_body(tok_hbm, pos_hbm, patch_hbm, out_hbm, idx_v, pos_stage,
             rows_v, post_v, trows_v, gsems, ssems)


def kernel(patches, token_table, pos_table):
    patches = patches.astype(jnp.int32)
    out_t = _sc_embed(token_table, pos_table, patches)
    return jnp.swapaxes(out_t, 1, 2)


# speedup vs baseline: 1.1054x; 1.1054x over previous
import functools

import jax
import jax.numpy as jnp
from jax import lax
from jax.experimental import pallas as pl
from jax.experimental.pallas import tpu as pltpu
from jax.experimental.pallas import tpu_sc as plsc

EMBED = 32
POS_V = 64
BATCH = 4096
SEQ = 128

NC, NS = 2, 16
NW = NC * NS
SEQ_PER_W = BATCH // NW
CHUNK = 4
NCHUNK = SEQ_PER_W // CHUNK
NBUF = 4
NLANE = 16
SBLK = SEQ // NLANE


def _transpose_add(rows_v, trows_v, pos_stage):
    lanes = lax.iota(jnp.int32, NLANE)

    @plsc.parallel_loop(0, SEQ)
    def body(s):
        svec = jnp.zeros((NLANE,), jnp.int32) + s
        ps = jnp.minimum(s, POS_V - 1)
        for h in range(EMBED // NLANE):
            sl = pl.ds(h * NLANE, NLANE)
            evec = lanes + h * NLANE
            p = pos_stage[ps, sl]
            for j in range(CHUNK):
                plsc.store_scatter(
                    trows_v.at[j], [evec, svec], rows_v[j, s, sl] + p)


def _sc_body(tok_hbm, pos_hbm, patch_hbm, out_hbm, idx_v, pos_stage,
             rows_v, trows_v, gsems, ssems):
    wid = lax.axis_index("s") * NC + lax.axis_index("c")
    seq0 = wid * SEQ_PER_W

    pltpu.sync_copy(patch_hbm.at[pl.ds(seq0, SEQ_PER_W)], idx_v)
    pltpu.sync_copy(pos_hbm, pos_stage)

    def issue_gathers(c, b):
        for j in range(CHUNK):
            pltpu.async_copy(
                tok_hbm.at[idx_v.at[c * CHUNK + j]],
                rows_v.at[b, j],
                gsems.at[b],
            )

    def wait_gathers(c, b):
        pltpu.make_async_copy(
            out_hbm.at[pl.ds(seq0, CHUNK)],
            rows_v.at[b],
            gsems.at[b],
        ).wait()

    issue_gathers(0, 0)
    issue_gathers(1, 1)

    def outer(c4, _):
        for u in range(NBUF):
            c = c4 * NBUF + u
            tb = u % 2

            @pl.when(c + 2 < NCHUNK)
            def _():
                issue_gathers(c + 2, (u + 2) % NBUF)

            wait_gathers(c, u)

            if u >= 2:
                _wait_store(out_hbm, trows_v, ssems, seq0, tb)
            else:
                @pl.when(c4 >= 1)
                def _():
                    _wait_store(out_hbm, trows_v, ssems, seq0, tb)

            _transpose_add(rows_v.at[u], trows_v.at[tb], pos_stage)
            pltpu.async_copy(
                trows_v.at[tb, :, :, pl.ds(0, SEQ)],
                out_hbm.at[pl.ds(seq0 + c * CHUNK, CHUNK)],
                ssems.at[tb],
            )
        return 0

    lax.fori_loop(0, NCHUNK // NBUF, outer, 0, unroll=False)
    for tb in range(2):
        _wait_store(out_hbm, trows_v, ssems, seq0, tb)


def _wait_store(out_hbm, trows_v, ssems, seq0, tb):
    pltpu.make_async_copy(
        trows_v.at[tb, :, :, pl.ds(0, SEQ)],
        out_hbm.at[pl.ds(seq0, CHUNK)],
        ssems.at[tb],
    ).wait()


@functools.partial(
    pl.kernel,
    out_type=jax.ShapeDtypeStruct((BATCH, EMBED, SEQ), jnp.float32),
    mesh=plsc.VectorSubcoreMesh(core_axis_name="c", subcore_axis_name="s"),
    scratch_types=[
        pltpu.VMEM((SEQ_PER_W, SEQ), jnp.int32),
        pltpu.VMEM((POS_V, EMBED), jnp.float32),
        pltpu.VMEM((NBUF, CHUNK, SEQ, EMBED), jnp.float32),
        pltpu.VMEM((2, CHUNK, EMBED, SEQ + 1), jnp.float32),
        pltpu.SemaphoreType.DMA((NBUF,)),
        pltpu.SemaphoreType.DMA((2,)),
    ],
    compiler_params=pltpu.CompilerParams(
        use_tc_tiling_on_sc=False, needs_layout_passes=False),
)
def _sc_embed(tok_hbm, pos_hbm, patch_hbm, out_hbm, idx_v, pos_stage,
              rows_v, trows_v, gsems, ssems):
    _sc_body(tok_hbm, pos_hbm, patch_hbm, out_hbm, idx_v, pos_stage,
             rows_v, trows_v, gsems, ssems)


def kernel(patches, token_table, pos_table):
    patches = patches.astype(jnp.int32)
    out_t = _sc_embed(token_table, pos_table, patches)
    return jnp.swapaxes(out_t, 1, 2)
